# Initial kernel scaffold; baseline (speedup 1.0000x reference)
import jax
import jax.numpy as jnp
from jax.experimental import pallas as pl


def kernel(features, edges, Ws, Wn, bs):
    raise NotImplementedError("write your pallas kernel here")


# speedup vs baseline: 3.0257x; 3.0257x over previous
import functools

import jax
import jax.numpy as jnp
from jax import lax
from jax.experimental import pallas as pl
from jax.experimental.pallas import tpu as pltpu
from jax.experimental.pallas import tpu_sc as plsc

NW = 32
NS = 16
CK = 128
BM = 1000


@functools.lru_cache(maxsize=None)
def _sc_agg_fn(n, d, ch, n_pad):
    mesh = plsc.VectorSubcoreMesh(core_axis_name="c", subcore_axis_name="s")
    nz = n_pad // NS

    def body(y_hbm, src_hbm, dst_hbm, z_hbm, out_hbm, src_v, dst_v, buf, shared, gsem):
        cid = lax.axis_index("c")
        sid = lax.axis_index("s")
        wid = cid * NS + sid
        pltpu.sync_copy(src_hbm.at[wid], src_v)
        pltpu.sync_copy(dst_hbm.at[wid], dst_v)
        pltpu.sync_copy(z_hbm.at[pl.ds(sid * nz, nz)],
                        shared.at[pl.ds(sid * nz, nz)])
        plsc.subcore_barrier()

        def step(j, carry):
            pltpu.async_copy(y_hbm.at[src_v.at[j]], buf, gsem).wait()
            pltpu.sync_copy(buf, shared.at[dst_v.at[j]], add=True)
            return carry

        lax.fori_loop(0, ch, step, 0)
        plsc.subcore_barrier()
        pltpu.sync_copy(shared.at[pl.ds(sid * nz, nz)],
                        out_hbm.at[cid].at[pl.ds(sid * nz, nz)])

    return pl.kernel(
        body,
        out_type=jax.ShapeDtypeStruct((2, n_pad, d), jnp.float32),
        mesh=mesh,
        scratch_types=[
            pltpu.VMEM((ch, CK), jnp.int32),
            pltpu.VMEM((ch, CK), jnp.int32),
            pltpu.VMEM((CK, d), jnp.float32),
            pltpu.VMEM_SHARED((n_pad, d), jnp.float32),
            pltpu.SemaphoreType.DMA,
        ],
    )


def _sc_agg(y, src_r, dst_r, zeros):
    n, d = y.shape
    ch = src_r.shape[1]
    return _sc_agg_fn(n, d, ch, zeros.shape[0])(y, src_r, dst_r, zeros)


def _row_spec(i_axis=True):
    return pl.BlockSpec((BM, 128), lambda i: (i, 0))


def _mm_first(x, w_self, w_neigh, b):
    m = x.shape[0]

    def body(x_ref, ws_ref, wn_ref, b_ref, xs_ref, y_ref):
        xb = x_ref[...]
        xs_ref[...] = (jnp.dot(xb, ws_ref[...], preferred_element_type=jnp.float32)
                       + b_ref[...])
        y_ref[...] = jnp.dot(xb, wn_ref[...], preferred_element_type=jnp.float32)

    return pl.pallas_call(
        body,
        grid=(m // BM,),
        in_specs=[
            pl.BlockSpec((BM, 128), lambda i: (i, 0)),
            pl.BlockSpec((128, 128), lambda i: (0, 0)),
            pl.BlockSpec((128, 128), lambda i: (0, 0)),
            pl.BlockSpec((1, 128), lambda i: (0, 0)),
        ],
        out_specs=[pl.BlockSpec((BM, 128), lambda i: (i, 0)),
                   pl.BlockSpec((BM, 128), lambda i: (i, 0))],
        out_shape=[jax.ShapeDtypeStruct((m, 128), jnp.float32)] * 2,
    )(x, w_self, w_neigh, b.reshape(1, 128))


def _mm_mid(xs_prev, agg, w_self, w_neigh, b, res=None):
    m = xs_prev.shape[0]
    with_res = res is not None

    def body(*refs):
        if with_res:
            xsp_ref, agg_ref, ws_ref, wn_ref, b_ref, r_ref, xs_ref, y_ref, x_ref = refs
        else:
            xsp_ref, agg_ref, ws_ref, wn_ref, b_ref, xs_ref, y_ref, x_ref = refs
        x = jnp.maximum(xsp_ref[...] + agg_ref[0] + agg_ref[1], 0.0)
        if with_res:
            x = x + r_ref[...]
        x_ref[...] = x
        xs_ref[...] = (jnp.dot(x, ws_ref[...], preferred_element_type=jnp.float32)
                       + b_ref[...])
        y_ref[...] = jnp.dot(x, wn_ref[...], preferred_element_type=jnp.float32)

    in_specs = [
        pl.BlockSpec((BM, 128), lambda i: (i, 0)),
        pl.BlockSpec((2, BM, 128), lambda i: (0, i, 0)),
        pl.BlockSpec((128, 128), lambda i: (0, 0)),
        pl.BlockSpec((128, 128), lambda i: (0, 0)),
        pl.BlockSpec((1, 128), lambda i: (0, 0)),
    ]
    args = [xs_prev, agg, w_self, w_neigh, b.reshape(1, 128)]
    if with_res:
        in_specs.append(pl.BlockSpec((BM, 128), lambda i: (i, 0)))
        args.append(res)
    return pl.pallas_call(
        body,
        grid=(m // BM,),
        in_specs=in_specs,
        out_specs=[pl.BlockSpec((BM, 128), lambda i: (i, 0))] * 3,
        out_shape=[jax.ShapeDtypeStruct((m, 128), jnp.float32)] * 3,
    )(*args)


def _combine_final(xs, agg):
    m = xs.shape[0]

    def body(xs_ref, agg_ref, o_ref):
        o_ref[...] = jnp.maximum(xs_ref[...] + agg_ref[0] + agg_ref[1], 0.0)

    return pl.pallas_call(
        body,
        grid=(m // BM,),
        in_specs=[pl.BlockSpec((BM, 128), lambda i: (i, 0)),
                  pl.BlockSpec((2, BM, 128), lambda i: (0, i, 0))],
        out_specs=pl.BlockSpec((BM, 128), lambda i: (i, 0)),
        out_shape=jax.ShapeDtypeStruct((m, 128), jnp.float32),
    )(xs, agg)


def kernel(features, edges, Ws, Wn, bs):
    n, d = features.shape
    e = edges.shape[1]
    ch = ((-(-e // (NW * CK)) + 7) // 8) * 8
    e_pad = NW * ch * CK
    n_pad = ((n + 1 + 127) // 128) * 128

    src = edges[0]
    dst = edges[1]
    pad = e_pad - e
    src_r = jnp.concatenate([src, jnp.zeros((pad,), jnp.int32)]).reshape(NW, ch, CK)
    dst_r = jnp.concatenate([dst, jnp.full((pad,), n, jnp.int32)]).reshape(NW, ch, CK)
    zeros = jnp.zeros((n_pad, d), jnp.float32)

    xs, y = _mm_first(features, Ws[0], Wn[0], bs[0])
    agg = _sc_agg(y, src_r, dst_r, zeros)
    h0 = None
    for l in range(1, 13):
        xs, y, x = _mm_mid(xs, agg, Ws[l], Wn[l], bs[l])
        if l == 1:
            h0 = x
        agg = _sc_agg(y, src_r, dst_r, zeros)
    xs, y, _ = _mm_mid(xs, agg, Ws[13], Wn[13], bs[13], res=h0)
    agg = _sc_agg(y, src_r, dst_r, zeros)
    return _combine_final(xs, agg)


# speedup vs baseline: 5.6310x; 1.8611x over previous
import functools

import jax
import jax.numpy as jnp
from jax import lax
from jax.experimental import pallas as pl
from jax.experimental.pallas import tpu as pltpu
from jax.experimental.pallas import tpu_sc as plsc

NW = 32
NS = 16
CK = 128
NBUF = 8
BM = 1000


@functools.lru_cache(maxsize=None)
def _sc_agg_fn(n, dh, ch, n_pad):
    mesh = plsc.VectorSubcoreMesh(core_axis_name="c", subcore_axis_name="s")
    nz = n_pad // NS
    grp = ch // NBUF

    def body(y_hbm, src_hbm, dst_hbm, z_hbm, out_hbm, src_v, dst_v, buf, shared,
             gsem, ssem):
        cid = lax.axis_index("c")
        sid = lax.axis_index("s")
        wid = cid * NS + sid
        pltpu.sync_copy(src_hbm.at[wid], src_v)
        pltpu.sync_copy(dst_hbm.at[wid], dst_v)
        pltpu.sync_copy(z_hbm.at[pl.ds(sid * nz, nz)],
                        shared.at[pl.ds(sid * nz, nz)])
        plsc.subcore_barrier()

        ytab = y_hbm.at[cid]
        for b in range(NBUF):
            pltpu.async_copy(ytab.at[src_v.at[b]], buf.at[b], gsem.at[b])

        def group(g, carry):
            for b in range(NBUF):
                j = g * NBUF + b
                pltpu.make_async_copy(ytab.at[src_v.at[j]], buf.at[b],
                                      gsem.at[b]).wait()
                pltpu.async_copy(buf.at[b], shared.at[dst_v.at[j]], ssem.at[b],
                                 add=True)
            for b in range(NBUF):
                j = g * NBUF + b
                pltpu.make_async_copy(buf.at[b], shared.at[dst_v.at[j]],
                                      ssem.at[b]).wait()

                @pl.when(g + 1 < grp)
                def _():
                    pltpu.async_copy(ytab.at[src_v.at[(g + 1) * NBUF + b]],
                                     buf.at[b], gsem.at[b])
            return carry

        lax.fori_loop(0, grp, group, 0)
        plsc.subcore_barrier()
        pltpu.sync_copy(shared.at[pl.ds(sid * nz, nz)],
                        out_hbm.at[cid].at[pl.ds(sid * nz, nz)])

    return pl.kernel(
        body,
        out_type=jax.ShapeDtypeStruct((2, n_pad, dh), jnp.float32),
        mesh=mesh,
        scratch_types=[
            pltpu.VMEM((ch, CK), jnp.int32),
            pltpu.VMEM((ch, CK), jnp.int32),
            pltpu.VMEM((NBUF, CK, dh), jnp.float32),
            pltpu.VMEM_SHARED((n_pad, dh), jnp.float32),
            pltpu.SemaphoreType.DMA((NBUF,)),
            pltpu.SemaphoreType.DMA((NBUF,)),
        ],
        compiler_params=pltpu.CompilerParams(use_tc_tiling_on_sc=False),
    )


def _sc_agg(y2, src_r, dst_r, zeros):
    _, n, dh = y2.shape
    ch = src_r.shape[1]
    return _sc_agg_fn(n, dh, ch, zeros.shape[0])(y2, src_r, dst_r, zeros)


def _mm_first(x, w_self, w_neigh, b):
    m, d = x.shape
    dh = d // 2

    def body(x_ref, ws_ref, wn_ref, b_ref, xs_ref, y_ref):
        xb = x_ref[...]
        xs_ref[...] = (jnp.dot(xb, ws_ref[...], preferred_element_type=jnp.float32)
                       + b_ref[...])
        y = jnp.dot(xb, wn_ref[...], preferred_element_type=jnp.float32)
        y_ref[0] = y[:, :dh]
        y_ref[1] = y[:, dh:]

    return pl.pallas_call(
        body,
        grid=(m // BM,),
        in_specs=[
            pl.BlockSpec((BM, d), lambda i: (i, 0)),
            pl.BlockSpec((d, d), lambda i: (0, 0)),
            pl.BlockSpec((d, d), lambda i: (0, 0)),
            pl.BlockSpec((1, d), lambda i: (0, 0)),
        ],
        out_specs=[pl.BlockSpec((BM, d), lambda i: (i, 0)),
                   pl.BlockSpec((2, BM, dh), lambda i: (0, i, 0))],
        out_shape=[jax.ShapeDtypeStruct((m, d), jnp.float32),
                   jax.ShapeDtypeStruct((2, m, dh), jnp.float32)],
    )(x, w_self, w_neigh, b.reshape(1, d))


def _mm_mid(xs_prev, agg, w_self, w_neigh, b, res=None):
    m, d = xs_prev.shape
    dh = d // 2
    n_pad = agg.shape[1]
    with_res = res is not None

    def body(*refs):
        if with_res:
            xsp_ref, agg_ref, ws_ref, wn_ref, b_ref, r_ref, xs_ref, y_ref, x_ref = refs
        else:
            xsp_ref, agg_ref, ws_ref, wn_ref, b_ref, xs_ref, y_ref, x_ref = refs
        a = jnp.concatenate([agg_ref[0], agg_ref[1]], axis=1)
        x = jnp.maximum(xsp_ref[...] + a, 0.0)
        if with_res:
            x = x + r_ref[...]
        x_ref[...] = x
        xs_ref[...] = (jnp.dot(x, ws_ref[...], preferred_element_type=jnp.float32)
                       + b_ref[...])
        y = jnp.dot(x, wn_ref[...], preferred_element_type=jnp.float32)
        y_ref[0] = y[:, :dh]
        y_ref[1] = y[:, dh:]

    in_specs = [
        pl.BlockSpec((BM, d), lambda i: (i, 0)),
        pl.BlockSpec((2, BM, dh), lambda i: (0, i, 0)),
        pl.BlockSpec((d, d), lambda i: (0, 0)),
        pl.BlockSpec((d, d), lambda i: (0, 0)),
        pl.BlockSpec((1, d), lambda i: (0, 0)),
    ]
    args = [xs_prev, agg, w_self, w_neigh, b.reshape(1, d)]
    if with_res:
        in_specs.append(pl.BlockSpec((BM, d), lambda i: (i, 0)))
        args.append(res)
    return pl.pallas_call(
        body,
        grid=(m // BM,),
        in_specs=in_specs,
        out_specs=[pl.BlockSpec((BM, d), lambda i: (i, 0)),
                   pl.BlockSpec((2, BM, dh), lambda i: (0, i, 0)),
                   pl.BlockSpec((BM, d), lambda i: (i, 0))],
        out_shape=[jax.ShapeDtypeStruct((m, d), jnp.float32),
                   jax.ShapeDtypeStruct((2, m, dh), jnp.float32),
                   jax.ShapeDtypeStruct((m, d), jnp.float32)],
    )(*args)


def _combine_final(xs, agg):
    m, d = xs.shape
    dh = d // 2

    def body(xs_ref, agg_ref, o_ref):
        a = jnp.concatenate([agg_ref[0], agg_ref[1]], axis=1)
        o_ref[...] = jnp.maximum(xs_ref[...] + a, 0.0)

    return pl.pallas_call(
        body,
        grid=(m // BM,),
        in_specs=[pl.BlockSpec((BM, d), lambda i: (i, 0)),
                  pl.BlockSpec((2, BM, dh), lambda i: (0, i, 0))],
        out_specs=pl.BlockSpec((BM, d), lambda i: (i, 0)),
        out_shape=jax.ShapeDtypeStruct((m, d), jnp.float32),
    )(xs, agg)


def kernel(features, edges, Ws, Wn, bs):
    n, d = features.shape
    e = edges.shape[1]
    ch = -(-e // (NW * CK))
    ch = -(-ch // NBUF) * NBUF
    e_pad = NW * ch * CK
    n_pad = ((n + 1 + 127) // 128) * 128

    src = edges[0]
    dst = edges[1]
    pad = e_pad - e
    src_r = jnp.concatenate([src, jnp.zeros((pad,), jnp.int32)]).reshape(NW, ch, CK)
    dst_r = jnp.concatenate([dst, jnp.full((pad,), n, jnp.int32)]).reshape(NW, ch, CK)
    zeros = jnp.zeros((n_pad, d // 2), jnp.float32)

    xs, y2 = _mm_first(features, Ws[0], Wn[0], bs[0])
    agg = _sc_agg(y2, src_r, dst_r, zeros)
    h0 = None
    for l in range(1, 13):
        xs, y2, x = _mm_mid(xs, agg, Ws[l], Wn[l], bs[l])
        if l == 1:
            h0 = x
        agg = _sc_agg(y2, src_r, dst_r, zeros)
    xs, y2, _ = _mm_mid(xs, agg, Ws[13], Wn[13], bs[13], res=h0)
    agg = _sc_agg(y2, src_r, dst_r, zeros)
    return _combine_final(xs, agg)
